# Initial kernel scaffold; baseline (speedup 1.0000x reference)
import jax
import jax.numpy as jnp
from jax.experimental import pallas as pl


def kernel(anchors_heats, corners_tl_regrs, corners_br_regrs):
    raise NotImplementedError("write your pallas kernel here")


# speedup vs baseline: 2.8071x; 2.8071x over previous
import jax
import jax.numpy as jnp
from jax.experimental import pallas as pl
from jax.experimental.pallas import tpu as pltpu

_B = 4
_C = 80
_H = 64
_W = 64
_HW = _H * _W
_CHW = _C * _HW
_L = 128
_R = _CHW // _L
_K = 2000
_ND = 1000
_HALF = 1.5
_MID = 2.25
_SCALE = 8.0
_ONE_BITS = 0x3F800000


def _iota(shape, dim, dtype=jnp.int32):
    out = jax.lax.broadcasted_iota(jnp.int32, shape, dim)
    return out if dtype == jnp.int32 else out.astype(dtype)


def _decode_kernel(heat_ref, maps_ref, out_ref):
    heat = heat_ref[...].reshape(_R, _L)
    bits = pltpu.bitcast(heat, jnp.int32)

    def body(_, carry):
        lo, hi = carry
        mid = lo + (hi - lo) // 2
        c = jnp.sum(jnp.where(bits > mid, 1.0, 0.0))
        small = c < _K
        new_lo = jnp.where(small, lo, mid + 1)
        new_hi = jnp.where(small, mid, hi)
        return new_lo, new_hi

    _, vstar = jax.lax.fori_loop(0, 31, body, (jnp.int32(0),
                                               jnp.int32(_ONE_BITS)))

    gt = bits > vstar
    eq = bits == vstar
    gtf = gt.astype(jnp.float32)
    eqf = eq.astype(jnp.float32)
    n_gt = jnp.sum(gtf)
    need_eq = _K - n_gt

    ltri = (_iota((_L, _L), 0) < _iota((_L, _L), 1)).astype(jnp.float32)
    rtri = (_iota((_R, _R), 1) < _iota((_R, _R), 0)).astype(jnp.float32)

    eq_lane = jnp.dot(eqf, ltri, preferred_element_type=jnp.float32, precision=jax.lax.Precision.HIGHEST)
    eq_base = jnp.dot(rtri, jnp.sum(eqf, axis=1, keepdims=True),
                      preferred_element_type=jnp.float32, precision=jax.lax.Precision.HIGHEST)
    eq_excl = eq_base + eq_lane

    sel = jnp.logical_or(gt, jnp.logical_and(eq, eq_excl < need_eq))
    self_f = sel.astype(jnp.float32)

    sel_lane = jnp.dot(self_f, ltri, preferred_element_type=jnp.float32, precision=jax.lax.Precision.HIGHEST)
    rowsum = jnp.sum(self_f, axis=1, keepdims=True)
    base = jnp.dot(rtri, rowsum, preferred_element_type=jnp.float32, precision=jax.lax.Precision.HIGHEST)
    p = base + sel_lane
    pm = jnp.where(sel, p, -1.0)

    kk = _iota((_R, _K), 1, jnp.float32)
    row_sel = jnp.logical_and(base <= kk, kk < base + rowsum)
    row_sel = row_sel.astype(jnp.float32)
    riota = _iota((_R, 1), 0, jnp.float32)
    rhs = jnp.concatenate([heat, pm, riota], axis=1)
    g = jax.lax.dot_general(row_sel, rhs, (((0,), (0,)), ((), ())),
                            preferred_element_type=jnp.float32, precision=jax.lax.Precision.HIGHEST)
    x_row = g[:, :_L]
    pm_row = g[:, _L:2 * _L]
    r_k = g[:, 2 * _L:2 * _L + 1]

    kcol = _iota((_K, 1), 0, jnp.float32)
    lane_sel = (pm_row == kcol).astype(jnp.float32)
    s = jnp.sum(lane_sel * x_row, axis=1, keepdims=True)
    lane = jnp.sum(lane_sel * _iota((_K, _L), 1, jnp.float32),
                   axis=1, keepdims=True)
    flat = (r_k * _L + lane).astype(jnp.int32)
    spatial = flat % _HW
    ys = (spatial // _W).astype(jnp.float32)
    xs = (spatial % _W).astype(jnp.float32)

    oh = (spatial == _iota((_K, _HW), 1)).astype(jnp.float32)
    maps = maps_ref[...].reshape(_HW, 4)
    reg = jnp.dot(oh, maps, preferred_element_type=jnp.float32, precision=jax.lax.Precision.HIGHEST)

    tl_x = xs - (_HALF * reg[:, 0:1] + _MID)
    tl_y = ys - (_HALF * reg[:, 1:2] + _MID)
    br_x = xs + (_HALF * reg[:, 2:3] + _MID)
    br_y = ys + (_HALF * reg[:, 3:4] + _MID)
    valid = jnp.logical_and(br_x >= tl_x, br_y >= tl_y)
    m = jnp.where(valid, s, -1.0)

    eyek = (_iota((_K, _K), 0) == _iota((_K, _K), 1)).astype(jnp.float32)
    ms = jnp.concatenate([m, s], axis=1)
    ms_rows = jax.lax.dot_general(ms, eyek, (((0,), (0,)), ((), ())),
                                  preferred_element_type=jnp.float32, precision=jax.lax.Precision.HIGHEST)
    m_row = ms_rows[0:1, :]
    s_row = ms_rows[1:2, :]
    ii = _iota((_K, _K), 0)
    jj = _iota((_K, _K), 1)
    beats = jnp.logical_or(
        m_row > m,
        jnp.logical_and(
            m_row == m,
            jnp.logical_or(s_row > s,
                           jnp.logical_and(s_row == s, jj < ii))))
    rank = jnp.sum(beats.astype(jnp.float32), axis=1, keepdims=True)

    rank_row = jax.lax.dot_general(rank, eyek, (((0,), (0,)), ((), ())),
                                   preferred_element_type=jnp.float32, precision=jax.lax.Precision.HIGHEST)
    oho = (rank_row == _iota((_ND, _K), 0, jnp.float32)).astype(jnp.float32)
    zeros = jnp.zeros_like(m)
    feats = jnp.concatenate(
        [m, _SCALE * tl_x, _SCALE * tl_y, _SCALE * br_x, _SCALE * br_y,
         zeros, zeros, zeros], axis=1)
    res = jnp.dot(oho, feats, preferred_element_type=jnp.float32, precision=jax.lax.Precision.HIGHEST)
    out_ref[...] = res[:, :7].reshape(1, _ND, 7)


def kernel(anchors_heats, corners_tl_regrs, corners_br_regrs):
    B = anchors_heats.shape[0]
    heat = anchors_heats.reshape(B, _R, _L)
    tl = jnp.transpose(corners_tl_regrs, (0, 2, 3, 1)).reshape(B, _HW, 2)
    br = jnp.transpose(corners_br_regrs, (0, 2, 3, 1)).reshape(B, _HW, 2)
    maps = jnp.concatenate([tl, br], axis=-1)

    return pl.pallas_call(
        _decode_kernel,
        grid=(B,),
        in_specs=[
            pl.BlockSpec((1, _R, _L), lambda b: (b, 0, 0)),
            pl.BlockSpec((1, _HW, 4), lambda b: (b, 0, 0)),
        ],
        out_specs=pl.BlockSpec((1, _ND, 7), lambda b: (b, 0, 0)),
        out_shape=jax.ShapeDtypeStruct((B, _ND, 7), jnp.float32),
    )(heat, maps)
